# baseline (speedup 1.0000x reference)
import functools

import jax
import jax.numpy as jnp
from jax import lax
from jax.experimental import pallas as pl
from jax.experimental.pallas import tpu as pltpu
from jax.experimental.pallas import tpu_sc as plsc

B = 16384
F = 64
NC = 2
NS = 16
NW = NC * NS
BPW = B // NW
CH = 128
NCH = BPW // CH
L = 16
G = BPW // L


def _make_sc_kernel():
    mesh = plsc.VectorSubcoreMesh(core_axis_name="c", subcore_axis_name="s")

    @functools.partial(
        pl.kernel,
        mesh=mesh,
        out_type=jax.ShapeDtypeStruct((B,), jnp.float32),
        compiler_params=pltpu.CompilerParams(
            needs_layout_passes=False, use_tc_tiling_on_sc=False),
        scratch_types=[
            pltpu.VMEM((NCH, CH), jnp.int32),
            pltpu.VMEM((NCH, CH), jnp.int32),
            pltpu.VMEM((BPW, F), jnp.float32),
            pltpu.VMEM((BPW, F), jnp.float32),
            pltpu.VMEM((BPW,), jnp.float32),
            pltpu.VMEM((BPW,), jnp.float32),
            pltpu.VMEM((L,), jnp.float32),
            pltpu.VMEM((BPW,), jnp.float32),
            pltpu.SemaphoreType.DMA,
        ],
    )
    def k(uidx_hbm, iidx_hbm, uf_hbm, if_hbm, ub_hbm, ib_hbm, gb_hbm,
          out_hbm, uidx_v, iidx_v, urows, vrows, ubv, ibv, gbv, outv, sem):
        wid = lax.axis_index("s") * NC + lax.axis_index("c")
        base = wid * BPW

        for j in range(NCH):
            pltpu.sync_copy(uidx_hbm.at[pl.ds(base + j * CH, CH)], uidx_v.at[j])
            pltpu.sync_copy(iidx_hbm.at[pl.ds(base + j * CH, CH)], iidx_v.at[j])
        pltpu.sync_copy(gb_hbm, gbv.at[pl.ds(0, 1)])

        copies = []
        for j in range(NCH):
            sl = pl.ds(j * CH, CH)
            copies.append(pltpu.async_copy(uf_hbm.at[uidx_v.at[j]], urows.at[sl], sem))
            copies.append(pltpu.async_copy(if_hbm.at[iidx_v.at[j]], vrows.at[sl], sem))
            copies.append(pltpu.async_copy(ub_hbm.at[uidx_v.at[j]], ubv.at[sl], sem))
            copies.append(pltpu.async_copy(ib_hbm.at[iidx_v.at[j]], ibv.at[sl], sem))
        for c in copies:
            c.wait()

        lanes = lax.iota(jnp.int32, L)
        zeros = jnp.zeros((L,), jnp.int32)
        gb = gbv[...][0]

        def body(g, _):
            rows = g * L + lanes
            row0 = pl.multiple_of(g * L, L)
            acc = ubv[pl.ds(row0, L)] + ibv[pl.ds(row0, L)] + gb
            for f in range(F):
                col = jnp.full((L,), f, jnp.int32)
                uu = plsc.load_gather(urows, [rows, col])
                vv = plsc.load_gather(vrows, [rows, col])
                acc = acc + uu * vv
            outv[pl.ds(row0, L)] = acc
            return 0

        lax.fori_loop(0, G, body, 0)

        pltpu.sync_copy(outv, out_hbm.at[pl.ds(base, BPW)])

    return k


_sc_kernel = _make_sc_kernel()


def kernel(user_idx, item_idx, user_factors, item_factors, user_bias,
           item_bias, global_bias):
    user_idx = user_idx.astype(jnp.int32)
    item_idx = item_idx.astype(jnp.int32)
    user_bias = user_bias.reshape(-1)
    item_bias = item_bias.reshape(-1)
    return _sc_kernel(user_idx, item_idx, user_factors, item_factors,
                      user_bias, item_bias, global_bias)
